# baseline (speedup 1.0000x reference)
import functools

import jax
import jax.numpy as jnp
from jax import lax
from jax.experimental import pallas as pl
from jax.experimental.pallas import tpu as pltpu
from jax.experimental.pallas import tpu_sc as plsc

_N = 16384
_D = 256
_K = 8192
_TN = 512
_WINDOWS = ((0, 2736), (2736, 5472), (5472, 8192))


def _argmin_body(x_ref, cb_ref, rn_ref, cbn0_ref, cbn1_ref, cbn2_ref,
                 idx_ref, msum_ref):
    step = pl.program_id(0)

    @pl.when(step == 0)
    def _():
        msum_ref[0, 0] = 0.0

    x = x_ref[...]
    rn = rn_ref[0, 0, :][:, None]
    cbns = (cbn0_ref, cbn1_ref, cbn2_ref)

    accv = jnp.full((_TN,), jnp.inf, jnp.float32)
    acct = jnp.zeros((_TN,), jnp.float32)
    acci = jnp.zeros((_TN,), jnp.int32)
    for w, (lo, hi) in enumerate(_WINDOWS):
        mm = lax.dot_general(x, cb_ref[lo:hi, :], (((1,), (1,)), ((), ())),
                             preferred_element_type=jnp.float32)
        d2 = (rn - 2.0 * mm) + cbns[w][...]
        m = d2.min(axis=1)
        a = jnp.argmin(d2, axis=1).astype(jnp.int32) + lo
        win = m < accv
        accv = jnp.where(win, m.astype(jnp.bfloat16).astype(jnp.float32), accv)
        acct = jnp.where(win, m, acct)
        acci = jnp.where(win, a, acci)

    idx_ref[0, 0, :] = acci
    msum_ref[0, 0] += jnp.sum(acct)


def _argmin_call(flat, codebook, rn, cbn_slices):
    n = flat.shape[0]
    nb = n // _TN
    return pl.pallas_call(
        _argmin_body,
        grid=(nb,),
        in_specs=[
            pl.BlockSpec((_TN, _D), lambda i: (i, 0)),
            pl.BlockSpec((_K, _D), lambda i: (0, 0)),
            pl.BlockSpec((1, 1, _TN), lambda i: (i, 0, 0)),
        ] + [pl.BlockSpec((1, hi - lo), lambda i: (0, 0))
             for lo, hi in _WINDOWS],
        out_specs=[
            pl.BlockSpec((1, 1, _TN), lambda i: (i, 0, 0)),
            pl.BlockSpec((1, 1), lambda i: (0, 0), memory_space=pltpu.SMEM),
        ],
        out_shape=[
            jax.ShapeDtypeStruct((nb, 1, _TN), jnp.int32),
            jax.ShapeDtypeStruct((1, 1), jnp.float32),
        ],
        compiler_params=pltpu.CompilerParams(
            dimension_semantics=("arbitrary",)),
    )(flat, codebook, rn.reshape(nb, 1, _TN), *cbn_slices)


_CH = 128


def _gather_call(codebook, idx):
    n = idx.shape[0]
    info = plsc.get_sparse_core_info()
    nw = info.num_cores * info.num_subcores
    b_per_w = n // nw
    nch = b_per_w // _CH
    mesh = plsc.VectorSubcoreMesh(core_axis_name="c", subcore_axis_name="s")

    @functools.partial(
        pl.kernel, mesh=mesh,
        out_type=jax.ShapeDtypeStruct((n, _D), jnp.float32),
        scratch_types=[
            pltpu.VMEM((nch, _CH), jnp.int32),
            pltpu.VMEM((_CH, _D), jnp.float32),
            pltpu.SemaphoreType.DMA,
        ],
    )
    def k(table_hbm, idx_hbm, out_hbm, idx_v, rows_v, sem):
        wid = lax.axis_index("s") * info.num_cores + lax.axis_index("c")
        base = wid * b_per_w
        pltpu.sync_copy(idx_hbm.at[pl.ds(wid * nch, nch)], idx_v)
        for c in range(nch):
            pltpu.async_copy(table_hbm.at[idx_v.at[c]], rows_v, sem).wait()
            pltpu.sync_copy(rows_v, out_hbm.at[pl.ds(base + c * _CH, _CH)])

    return k(codebook, idx.reshape(n // _CH, _CH))


def kernel(latent, codebook):
    b, c, h, w = latent.shape
    flat = jnp.transpose(latent, (0, 2, 3, 1)).reshape(-1, c)
    rn = jnp.sum(flat * flat, axis=1)
    cbn = jnp.sum(codebook * codebook, axis=1).reshape(1, _K)
    cbn_slices = [cbn[:, lo:hi] for lo, hi in _WINDOWS]

    half = _N // 2
    qs, msums = [], []
    for s in (0, half):
        idx3, msum = _argmin_call(flat[s:s + half], codebook,
                                  rn[s:s + half], cbn_slices)
        qs.append(_gather_call(codebook, idx3.reshape(-1)))
        msums.append(msum[0, 0])

    q = jnp.concatenate(qs, axis=0)
    loss = (msums[0] + msums[1]) / jnp.float32(_N * _D)
    out = q.reshape(b, h, w, c).transpose(0, 3, 1, 2)
    return out, loss, loss


# speedup vs baseline: 1.1171x; 1.1171x over previous
import functools

import jax
import jax.numpy as jnp
from jax import lax
from jax.experimental import pallas as pl
from jax.experimental.pallas import tpu as pltpu
from jax.experimental.pallas import tpu_sc as plsc

_N = 16384
_D = 256
_K = 8192
_TN = 512
_WINDOWS = ((0, 2736), (2736, 5472), (5472, 8192))


def _argmin_body(x_ref, cb_ref, rn_ref, cbn0_ref, cbn1_ref, cbn2_ref,
                 idx_ref, msum_ref):
    step = pl.program_id(0)

    @pl.when(step == 0)
    def _():
        msum_ref[0, 0] = 0.0

    x = x_ref[...]
    rn = rn_ref[0, 0, :][:, None]
    cbns = (cbn0_ref, cbn1_ref, cbn2_ref)

    accv = jnp.full((_TN,), jnp.inf, jnp.float32)
    acct = jnp.zeros((_TN,), jnp.float32)
    acci = jnp.zeros((_TN,), jnp.int32)
    for w, (lo, hi) in enumerate(_WINDOWS):
        mm = lax.dot_general(x, cb_ref[lo:hi, :], (((1,), (1,)), ((), ())),
                             preferred_element_type=jnp.float32)
        d2 = (rn - 2.0 * mm) + cbns[w][...]
        m = d2.min(axis=1)
        a = jnp.argmin(d2, axis=1).astype(jnp.int32) + lo
        win = m < accv
        accv = jnp.where(win, m.astype(jnp.bfloat16).astype(jnp.float32), accv)
        acct = jnp.where(win, m, acct)
        acci = jnp.where(win, a, acci)

    idx_ref[0, 0, :] = acci
    msum_ref[0, 0] += jnp.sum(acct)


def _argmin_call(flat, codebook, rn, cbn_slices):
    n = flat.shape[0]
    nb = n // _TN
    return pl.pallas_call(
        _argmin_body,
        grid=(nb,),
        in_specs=[
            pl.BlockSpec((_TN, _D), lambda i: (i, 0)),
            pl.BlockSpec((_K, _D), lambda i: (0, 0)),
            pl.BlockSpec((1, 1, _TN), lambda i: (i, 0, 0)),
        ] + [pl.BlockSpec((1, hi - lo), lambda i: (0, 0))
             for lo, hi in _WINDOWS],
        out_specs=[
            pl.BlockSpec((1, 1, _TN), lambda i: (i, 0, 0)),
            pl.BlockSpec((1, 1), lambda i: (0, 0), memory_space=pltpu.SMEM),
        ],
        out_shape=[
            jax.ShapeDtypeStruct((nb, 1, _TN), jnp.int32),
            jax.ShapeDtypeStruct((1, 1), jnp.float32),
        ],
        compiler_params=pltpu.CompilerParams(
            dimension_semantics=("arbitrary",)),
    )(flat, codebook, rn.reshape(nb, 1, _TN), *cbn_slices)


_CH = 128


def _gather_call(codebook, idx):
    n = idx.shape[0]
    info = plsc.get_sparse_core_info()
    nw = info.num_cores * info.num_subcores
    b_per_w = n // nw
    nch = b_per_w // _CH
    mesh = plsc.VectorSubcoreMesh(core_axis_name="c", subcore_axis_name="s")

    @functools.partial(
        pl.kernel, mesh=mesh,
        out_type=jax.ShapeDtypeStruct((n, _D), jnp.float32),
        scratch_types=[
            pltpu.VMEM((nch, _CH), jnp.int32),
            pltpu.VMEM((_CH, _D), jnp.float32),
            pltpu.SemaphoreType.DMA,
        ],
    )
    def k(table_hbm, idx_hbm, out_hbm, idx_v, rows_v, sem):
        wid = lax.axis_index("s") * info.num_cores + lax.axis_index("c")
        base = wid * b_per_w
        pltpu.sync_copy(idx_hbm.at[pl.ds(wid * nch, nch)], idx_v)
        for c in range(nch):
            pltpu.async_copy(table_hbm.at[idx_v.at[c]], rows_v, sem).wait()
            pltpu.sync_copy(rows_v, out_hbm.at[pl.ds(base + c * _CH, _CH)])

    return k(codebook, idx.reshape(n // _CH, _CH))


def kernel(latent, codebook):
    b, c, h, w = latent.shape
    flat = jnp.transpose(latent, (0, 2, 3, 1)).reshape(-1, c)
    rn = jnp.sum(flat * flat, axis=1)
    cbn = jnp.sum(codebook * codebook, axis=1).reshape(1, _K)
    cbn_slices = [cbn[:, lo:hi] for lo, hi in _WINDOWS]

    idx3, msum = _argmin_call(flat, codebook, rn, cbn_slices)
    q = _gather_call(codebook, idx3.reshape(-1))
    loss = msum[0, 0] / jnp.float32(_N * _D)
    out = q.reshape(b, h, w, c).transpose(0, 3, 1, 2)
    return out, loss, loss
